# Initial kernel scaffold; baseline (speedup 1.0000x reference)
import jax
import jax.numpy as jnp
from jax.experimental import pallas as pl


def kernel(tokens, W):
    raise NotImplementedError("write your pallas kernel here")


# speedup vs baseline: 2.9103x; 2.9103x over previous
import functools
import math

import jax
import jax.numpy as jnp
from jax import lax
from jax.experimental import pallas as pl
from jax.experimental.pallas import tpu as pltpu
from jax.experimental.pallas import tpu_sc as plsc

VOCAB_ROWS = 100000
D = 128
B_TOK = 1024
S_TOK = 200
N_IDX = B_TOK * S_TOK
CHUNK = 128
SCALE = math.sqrt(float(D))


def _make_sc_kernel():
    info = plsc.get_sparse_core_info()
    nc, ns = info.num_cores, info.num_subcores
    nw = nc * ns
    chunks_per_w = N_IDX // (nw * CHUNK)
    nbuf = 2
    assert chunks_per_w % nbuf == 0

    mesh = plsc.VectorSubcoreMesh(core_axis_name="c", subcore_axis_name="s")

    @functools.partial(
        pl.kernel,
        mesh=mesh,
        out_type=jax.ShapeDtypeStruct((N_IDX, D), jnp.float32),
        scratch_types=[
            pltpu.VMEM((chunks_per_w, CHUNK), jnp.int32),
            pltpu.VMEM((CHUNK, D), jnp.float32),
            pltpu.VMEM((CHUNK, D), jnp.float32),
            pltpu.VMEM((CHUNK, D), jnp.float32),
            pltpu.VMEM((CHUNK, D), jnp.float32),
            pltpu.SemaphoreType.DMA,
            pltpu.SemaphoreType.DMA,
            pltpu.SemaphoreType.DMA,
            pltpu.SemaphoreType.DMA,
        ],
    )
    def emb(w_hbm, idx_hbm, out_hbm, idx_v, g0, g1, s0, s1,
            gsem0, gsem1, ssem0, ssem1):
        wid = lax.axis_index("s") * nc + lax.axis_index("c")
        chunk0 = wid * chunks_per_w
        gbufs = (g0, g1)
        sbufs = (s0, s1)
        gsems = (gsem0, gsem1)
        ssems = (ssem0, ssem1)

        pltpu.sync_copy(idx_hbm.at[wid], idx_v)

        for b in range(nbuf):
            pltpu.async_copy(w_hbm.at[idx_v.at[b]], gbufs[b], gsems[b])

        def group(g, carry):
            for b in range(nbuf):
                cl = g * nbuf + b
                row0 = (chunk0 + cl) * CHUNK
                gbuf, sbuf = gbufs[b], sbufs[b]
                gsem, ssem = gsems[b], ssems[b]

                pltpu.make_async_copy(w_hbm.at[idx_v.at[cl]], gbuf, gsem).wait()

                @pl.when(cl >= nbuf)
                def _():
                    pltpu.make_async_copy(
                        sbuf, out_hbm.at[pl.ds(row0, CHUNK)], ssem).wait()

                def srow(i, c):
                    for j in range(D // 16):
                        sbuf[i, pl.ds(j * 16, 16)] = (
                            gbuf[i, pl.ds(j * 16, 16)] * SCALE)
                    return c
                lax.fori_loop(0, CHUNK, srow, 0, unroll=2)

                pltpu.async_copy(sbuf, out_hbm.at[pl.ds(row0, CHUNK)], ssem)

                @pl.when(cl + nbuf < chunks_per_w)
                def _():
                    pltpu.async_copy(
                        w_hbm.at[idx_v.at[cl + nbuf]], gbuf, gsem)
            return carry

        lax.fori_loop(0, chunks_per_w // nbuf, group, 0)

        for b in range(nbuf):
            pltpu.make_async_copy(
                sbufs[b], out_hbm.at[pl.ds(0, CHUNK)], ssems[b]).wait()

    return emb


def kernel(tokens, W):
    nw = 32
    idx = tokens.reshape(-1).astype(jnp.int32).reshape(
        nw, N_IDX // (nw * CHUNK), CHUNK)
    out = _make_sc_kernel()(W, idx)
    return out.reshape(B_TOK, S_TOK, D)


# speedup vs baseline: 2.9247x; 1.0050x over previous
import functools
import math

import jax
import jax.numpy as jnp
from jax import lax
from jax.experimental import pallas as pl
from jax.experimental.pallas import tpu as pltpu
from jax.experimental.pallas import tpu_sc as plsc

VOCAB_ROWS = 100000
D = 128
B_TOK = 1024
S_TOK = 200
N_IDX = B_TOK * S_TOK
CHUNK = 128
SCALE = math.sqrt(float(D))


def _make_sc_kernel():
    info = plsc.get_sparse_core_info()
    nc, ns = info.num_cores, info.num_subcores
    nw = nc * ns
    chunks_per_w = N_IDX // (nw * CHUNK)
    nbuf = 2
    assert chunks_per_w % nbuf == 0

    mesh = plsc.VectorSubcoreMesh(core_axis_name="c", subcore_axis_name="s")

    @functools.partial(
        pl.kernel,
        mesh=mesh,
        out_type=jax.ShapeDtypeStruct((N_IDX, D), jnp.float32),
        scratch_types=[
            pltpu.VMEM((chunks_per_w, CHUNK), jnp.int32),
            pltpu.VMEM((CHUNK, D), jnp.float32),
            pltpu.VMEM((CHUNK, D), jnp.float32),
            pltpu.VMEM((CHUNK, D), jnp.float32),
            pltpu.VMEM((CHUNK, D), jnp.float32),
            pltpu.SemaphoreType.DMA,
            pltpu.SemaphoreType.DMA,
            pltpu.SemaphoreType.DMA,
            pltpu.SemaphoreType.DMA,
        ],
    )
    def emb(w_hbm, idx_hbm, out_hbm, idx_v, g0, g1, s0, s1,
            gsem0, gsem1, ssem0, ssem1):
        wid = lax.axis_index("s") * nc + lax.axis_index("c")
        chunk0 = wid * chunks_per_w
        gbufs = (g0, g1)
        sbufs = (s0, s1)
        gsems = (gsem0, gsem1)
        ssems = (ssem0, ssem1)

        pltpu.sync_copy(idx_hbm.at[wid], idx_v)

        for b in range(nbuf):
            pltpu.async_copy(w_hbm.at[idx_v.at[b]], gbufs[b], gsems[b])

        def group(g, carry):
            for b in range(nbuf):
                cl = g * nbuf + b
                row0 = (chunk0 + cl) * CHUNK
                gbuf, sbuf = gbufs[b], sbufs[b]
                gsem, ssem = gsems[b], ssems[b]

                pltpu.make_async_copy(w_hbm.at[idx_v.at[cl]], gbuf, gsem).wait()

                @pl.when(cl >= nbuf)
                def _():
                    pltpu.make_async_copy(
                        sbuf, out_hbm.at[pl.ds(row0, CHUNK)], ssem).wait()

                def srow(i, c):
                    for j in range(D // 16):
                        sbuf[i, pl.ds(j * 16, 16)] = (
                            gbuf[i, pl.ds(j * 16, 16)] * SCALE)
                    return c
                lax.fori_loop(0, CHUNK, srow, 0, unroll=8)

                pltpu.async_copy(sbuf, out_hbm.at[pl.ds(row0, CHUNK)], ssem)

                @pl.when(cl + nbuf < chunks_per_w)
                def _():
                    pltpu.async_copy(
                        w_hbm.at[idx_v.at[cl + nbuf]], gbuf, gsem)
            return carry

        lax.fori_loop(0, chunks_per_w // nbuf, group, 0)

        for b in range(nbuf):
            pltpu.make_async_copy(
                sbufs[b], out_hbm.at[pl.ds(0, CHUNK)], ssems[b]).wait()

    return emb


def kernel(tokens, W):
    nw = 32
    idx = tokens.reshape(-1).astype(jnp.int32).reshape(
        nw, N_IDX // (nw * CHUNK), CHUNK)
    out = _make_sc_kernel()(W, idx)
    return out.reshape(B_TOK, S_TOK, D)
